# baseline (speedup 1.0000x reference)
import math

import jax
import jax.numpy as jnp
from jax import lax
from jax.experimental import pallas as pl
from jax.experimental.pallas import tpu as pltpu
from jax.experimental.pallas import tpu_sc as plsc

_NUM_NEG = 29
_NCAND = _NUM_NEG + 1
_KS = (1, 5, 10)
_LANES = 16
_NUM_CORES = 2
_NUM_SUBCORES = 16
_NW = _NUM_CORES * _NUM_SUBCORES
_CHUNK = 120


def _sc_stats_call(B):
    gp_w = B // _LANES // _NW
    per_w = gp_w * _NCAND * _LANES
    n_chunks = per_w // _CHUNK
    assert per_w % _CHUNK == 0
    mesh = plsc.VectorSubcoreMesh(
        core_axis_name="c", subcore_axis_name="s",
        num_cores=_NUM_CORES, num_subcores=_NUM_SUBCORES)

    def body(scores_hbm, idx_hbm, out_hbm, idx_v, vals_v, stat_v, sem):
        wid = lax.axis_index("s") * _NUM_CORES + lax.axis_index("c")
        base = wid * per_w
        pltpu.sync_copy(idx_hbm.at[pl.ds(base, per_w)], idx_v)
        copies = [
            pltpu.async_copy(
                scores_hbm.at[idx_v.at[pl.ds(ch * _CHUNK, _CHUNK)]],
                vals_v.at[pl.ds(ch * _CHUNK, _CHUNK)], sem)
            for ch in range(n_chunks)
        ]
        for c in copies:
            c.wait()
        for gl in range(gp_w):
            goff = gl * _NCAND * _LANES
            vs = [vals_v[pl.ds(goff + j * _LANES, _LANES)]
                  for j in range(_NCAND)]
            x0 = vs[0]
            mx = x0
            rank = jnp.zeros((_LANES,), jnp.float32)
            one = jnp.ones((_LANES,), jnp.float32)
            zero = jnp.zeros((_LANES,), jnp.float32)
            for j in range(1, _NCAND):
                rank = rank + jnp.where(vs[j] > x0, one, zero)
                mx = jnp.maximum(mx, vs[j])
            se = jnp.zeros((_LANES,), jnp.float32)
            for j in range(_NCAND):
                se = se + jnp.exp(vs[j] - mx)
            stat_v[0, pl.ds(gl * _LANES, _LANES)] = rank
            stat_v[1, pl.ds(gl * _LANES, _LANES)] = se
            stat_v[2, pl.ds(gl * _LANES, _LANES)] = mx - x0
        span = gp_w * _LANES
        for c in range(3):
            pltpu.sync_copy(stat_v.at[c],
                            out_hbm.at[c, pl.ds(wid * span, span)])

    return pl.kernel(
        body,
        out_type=jax.ShapeDtypeStruct((3, B), jnp.float32),
        mesh=mesh,
        scratch_types=[
            pltpu.VMEM((per_w,), jnp.int32),
            pltpu.VMEM((per_w,), jnp.float32),
            pltpu.VMEM((3, gp_w * _LANES), jnp.float32),
            pltpu.SemaphoreType.DMA,
        ],
    )


def _tc_reduce_call(stats3, B):
    inv = 1.0 / B
    ln2 = math.log(2.0)

    def body(st_ref, out_ref):
        r = st_ref[0]
        se = st_ref[1]
        d = st_ref[2]
        out_ref[7] = jnp.sum(jnp.log(se) + d) * inv
        w = ln2 / jnp.log(r + 2.0)
        for i, k in enumerate(_KS):
            hit = jnp.where(r < k, 1.0, 0.0)
            out_ref[2 * i] = jnp.sum(hit) * inv
            out_ref[2 * i + 1] = jnp.sum(hit * w) * inv
        out_ref[6] = jnp.sum(1.0 / (r + 1.0)) * inv

    return pl.pallas_call(
        body,
        out_shape=jax.ShapeDtypeStruct((8,), jnp.float32),
        out_specs=pl.BlockSpec(memory_space=pltpu.SMEM),
    )(stats3)


def kernel(scores, labels):
    B, C = scores.shape
    m = labels.shape[1]
    key = jax.random.key(42)
    sorted_lab = jnp.sort(labels, axis=1)
    negs = jax.random.randint(key, (B, _NUM_NEG), 0, C - m, dtype=labels.dtype)
    for j in range(m):
        e = sorted_lab[:, j:j + 1]
        negs = negs + (negs >= e).astype(negs.dtype)
    cand = jnp.concatenate([labels[:, -1:], negs], axis=1)
    rowbase = (jnp.arange(B, dtype=jnp.int32) * C)[:, None]
    flat = (cand + rowbase).reshape(B // _LANES, _LANES, _NCAND)
    idx = flat.transpose(0, 2, 1).reshape(-1)
    stats = _sc_stats_call(B)(scores.reshape(-1), idx)
    return _tc_reduce_call(stats.reshape(3, B // 128, 128), B)


# speedup vs baseline: 26.4054x; 26.4054x over previous
import math

import jax
import jax.numpy as jnp
from jax import lax
from jax.experimental import pallas as pl
from jax.experimental.pallas import tpu as pltpu
from jax.experimental.pallas import tpu_sc as plsc

_NUM_NEG = 29
_NCAND = _NUM_NEG + 1
_KS = (1, 5, 10)
_LANES = 16
_NUM_CORES = 2
_NUM_SUBCORES = 16
_NW = _NUM_CORES * _NUM_SUBCORES
_CHUNK = 120


def _sc_stats_call(B):
    gp_w = B // _LANES // _NW
    per_w = gp_w * _NCAND * _LANES
    n_chunks = per_w // _CHUNK
    assert per_w % _CHUNK == 0
    mesh = plsc.VectorSubcoreMesh(
        core_axis_name="c", subcore_axis_name="s",
        num_cores=_NUM_CORES, num_subcores=_NUM_SUBCORES)

    def body(scores_hbm, idx_hbm, out_hbm, idx_v, vals_v, stat_v, sem):
        wid = lax.axis_index("s") * _NUM_CORES + lax.axis_index("c")
        base = wid * per_w
        pltpu.sync_copy(idx_hbm.at[pl.ds(base, per_w)], idx_v)
        copies = [
            pltpu.async_copy(
                scores_hbm.at[idx_v.at[pl.ds(ch * _CHUNK, _CHUNK)]],
                vals_v.at[pl.ds(ch * _CHUNK, _CHUNK)], sem)
            for ch in range(n_chunks)
        ]
        for c in copies:
            c.wait()
        for gl in range(gp_w):
            goff = gl * _NCAND * _LANES
            vs = [vals_v[pl.ds(goff + j * _LANES, _LANES)]
                  for j in range(_NCAND)]
            x0 = vs[0]
            mx = x0
            rank = jnp.zeros((_LANES,), jnp.float32)
            one = jnp.ones((_LANES,), jnp.float32)
            zero = jnp.zeros((_LANES,), jnp.float32)
            for j in range(1, _NCAND):
                rank = rank + jnp.where(vs[j] > x0, one, zero)
                mx = jnp.maximum(mx, vs[j])
            se = jnp.zeros((_LANES,), jnp.float32)
            for j in range(_NCAND):
                se = se + jnp.exp(vs[j] - mx)
            stat_v[0, pl.ds(gl * _LANES, _LANES)] = rank
            stat_v[1, pl.ds(gl * _LANES, _LANES)] = se
            stat_v[2, pl.ds(gl * _LANES, _LANES)] = mx - x0
        span = gp_w * _LANES
        for c in range(3):
            pltpu.sync_copy(stat_v.at[c],
                            out_hbm.at[c, pl.ds(wid * span, span)])

    return pl.kernel(
        body,
        out_type=jax.ShapeDtypeStruct((3, B), jnp.float32),
        mesh=mesh,
        scratch_types=[
            pltpu.VMEM((per_w,), jnp.int32),
            pltpu.VMEM((per_w,), jnp.float32),
            pltpu.VMEM((3, gp_w * _LANES), jnp.float32),
            pltpu.SemaphoreType.DMA,
        ],
    )


def _tc_reduce_call(stats3, B):
    inv = 1.0 / B
    ln2 = math.log(2.0)

    def body(st_ref, out_ref):
        r = st_ref[0]
        se = st_ref[1]
        d = st_ref[2]
        out_ref[7] = jnp.sum(jnp.log(se) + d) * inv
        w = ln2 / jnp.log(r + 2.0)
        for i, k in enumerate(_KS):
            hit = jnp.where(r < k, 1.0, 0.0)
            out_ref[2 * i] = jnp.sum(hit) * inv
            out_ref[2 * i + 1] = jnp.sum(hit * w) * inv
        out_ref[6] = jnp.sum(1.0 / (r + 1.0)) * inv

    return pl.pallas_call(
        body,
        out_shape=jax.ShapeDtypeStruct((8,), jnp.float32),
        out_specs=pl.BlockSpec(memory_space=pltpu.SMEM),
    )(stats3)


def kernel(scores, labels):
    B, C = scores.shape
    m = labels.shape[1]
    key = jax.random.key(42)
    sorted_lab = jnp.sort(labels, axis=1)
    negs = jax.random.randint(key, (B, _NUM_NEG), 0, C - m, dtype=labels.dtype)
    for j in range(m):
        e = sorted_lab[:, j:j + 1]
        negs = negs + (negs >= e).astype(negs.dtype)
    cand = jnp.concatenate([labels[:, -1:], negs], axis=1)
    b = jnp.arange(B, dtype=jnp.int32)[:, None]
    poff = ((cand // 8) * ((B // 128) * 1024) + (b // 128) * 1024
            + (cand % 8) * 128 + (b % 128))
    flat = poff.reshape(B // _LANES, _LANES, _NCAND)
    idx = flat.transpose(0, 2, 1).reshape(-1)
    phys = (scores.reshape(B // 128, 128, C // 8, 8)
            .transpose(2, 0, 3, 1).reshape(-1))
    stats = _sc_stats_call(B)(phys, idx)
    return _tc_reduce_call(stats.reshape(3, B // 128, 128), B)


# speedup vs baseline: 28.7585x; 1.0891x over previous
import functools
import math

import jax
import jax.numpy as jnp
import numpy as np
from jax import lax
from jax.experimental import pallas as pl
from jax.experimental.pallas import tpu as pltpu
from jax.experimental.pallas import tpu_sc as plsc

_NUM_NEG = 29
_NCAND = _NUM_NEG + 1
_KS = (1, 5, 10)
_LANES = 16
_NUM_CORES = 2
_NUM_SUBCORES = 16
_NW = _NUM_CORES * _NUM_SUBCORES
_CHUNK = 120


_ROT = ((13, 15, 26, 6), (17, 29, 16, 24))


def _tf2x32(k1, k2, x1, x2):
    u32 = np.uint32
    ks = (u32(k1), u32(k2), u32(u32(k1) ^ u32(k2) ^ u32(0x1BD11BDA)))
    x0 = (x1 + ks[0]).astype(np.uint32)
    x1 = (x2 + ks[1]).astype(np.uint32)
    sched = ((0, 1, 2, 1), (1, 2, 0, 2), (0, 0, 1, 3), (1, 1, 2, 4),
             (0, 2, 0, 5))
    for g, a, b, inc in sched:
        for r in _ROT[g]:
            x0 = (x0 + x1).astype(np.uint32)
            x1 = x0 ^ ((x1 << u32(r)) | (x1 >> u32(32 - r)))
        x0 = (x0 + ks[a]).astype(np.uint32)
        x1 = (x1 + ks[b] + u32(inc)).astype(np.uint32)
    return x0, x1


@functools.lru_cache(maxsize=None)
def _negs_const(B, C, m):
    n = B * _NUM_NEG
    b1, b2 = _tf2x32(0, 42, np.zeros(2, np.uint32),
                     np.arange(2, dtype=np.uint32))
    cnt = np.arange(n, dtype=np.uint64)
    chi = (cnt >> np.uint64(32)).astype(np.uint32)
    clo = (cnt & np.uint64(0xFFFFFFFF)).astype(np.uint32)
    h1, h2 = _tf2x32(b1[0], b2[0], chi, clo)
    l1, l2 = _tf2x32(b1[1], b2[1], chi, clo)
    higher, lower = h1 ^ h2, l1 ^ l2
    span = np.uint32(C - m)
    mult = int(np.uint32(65536) % span)
    mult = np.uint32((mult * mult) % (1 << 32) % int(span))
    off = ((higher % span) * mult + (lower % span)).astype(np.uint32) % span
    negs = off.astype(np.int32).reshape(B, _NUM_NEG)
    return np.ascontiguousarray(
        negs.reshape(B // _LANES, _LANES, _NUM_NEG).transpose(0, 2, 1)
        .reshape(-1))


def _sc_stats_call(B, C, m):
    gp_w = B // _LANES // _NW
    per_w = gp_w * _NCAND * _LANES
    nper_w = gp_w * _NUM_NEG * _LANES
    n_chunks = per_w // _CHUNK
    assert per_w % _CHUNK == 0
    trow = (B // 128) * 1024
    mesh = plsc.VectorSubcoreMesh(
        core_axis_name="c", subcore_axis_name="s",
        num_cores=_NUM_CORES, num_subcores=_NUM_SUBCORES)

    def body(scores_hbm, labt_hbm, negs_hbm, out_hbm,
             lab_v, negs_v, idx_v, vals_v, stat_v, sem, gsem):
        wid = lax.axis_index("s") * _NUM_CORES + lax.axis_index("c")
        span = gp_w * _LANES
        lcopies = [
            pltpu.async_copy(labt_hbm.at[pl.ds(t * B + wid * span, span)],
                             lab_v.at[t], sem)
            for t in range(m)
        ]
        cn = pltpu.async_copy(
            negs_hbm.at[pl.ds(wid * nper_w, nper_w)], negs_v, sem)
        for cc in lcopies:
            cc.wait()
        cn.wait()
        for gl in range(gp_w):
            goff = gl * _LANES
            b = (wid * (gp_w * _LANES) + goff
                 + lax.iota(jnp.int32, _LANES))
            b_term = ((b >> 7) << 10) + (b & 127)
            svs = [lab_v[t, pl.ds(goff, _LANES)] for t in range(m)]
            x0c = svs[m - 1]
            for rnd in range(m):
                start = rnd & 1
                for t in range(start, m - 1, 2):
                    lo = jnp.minimum(svs[t], svs[t + 1])
                    hi = jnp.maximum(svs[t], svs[t + 1])
                    svs[t], svs[t + 1] = lo, hi
            one = jnp.ones((_LANES,), jnp.int32)
            zero = jnp.zeros((_LANES,), jnp.int32)
            nvs = [negs_v[pl.ds(gl * (_NUM_NEG * _LANES) + j * _LANES,
                                _LANES)] for j in range(_NUM_NEG)]
            for t in range(m):
                e = svs[t]
                for j in range(_NUM_NEG):
                    nvs[j] = nvs[j] + jnp.where(nvs[j] >= e, one, zero)
            cands = [x0c] + nvs
            ibase = gl * (_NCAND * _LANES)
            for j in range(_NCAND):
                c = cands[j]
                idx_v[pl.ds(ibase + j * _LANES, _LANES)] = (
                    ((c >> 3) * trow) + ((c & 7) << 7) + b_term)
        copies = [
            pltpu.async_copy(
                scores_hbm.at[idx_v.at[pl.ds(ch * _CHUNK, _CHUNK)]],
                vals_v.at[pl.ds(ch * _CHUNK, _CHUNK)], gsem)
            for ch in range(n_chunks)
        ]
        for cc in copies:
            cc.wait()
        for gl in range(gp_w):
            goff = gl * (_NCAND * _LANES)
            vs = [vals_v[pl.ds(goff + j * _LANES, _LANES)]
                  for j in range(_NCAND)]
            x0 = vs[0]
            mx = x0
            rank = jnp.zeros((_LANES,), jnp.float32)
            fone = jnp.ones((_LANES,), jnp.float32)
            fzero = jnp.zeros((_LANES,), jnp.float32)
            for j in range(1, _NCAND):
                rank = rank + jnp.where(vs[j] > x0, fone, fzero)
                mx = jnp.maximum(mx, vs[j])
            se = jnp.zeros((_LANES,), jnp.float32)
            for j in range(_NCAND):
                se = se + jnp.exp(vs[j] - mx)
            stat_v[0, pl.ds(gl * _LANES, _LANES)] = rank
            stat_v[1, pl.ds(gl * _LANES, _LANES)] = se
            stat_v[2, pl.ds(gl * _LANES, _LANES)] = mx - x0
        for c in range(3):
            pltpu.sync_copy(stat_v.at[c],
                            out_hbm.at[c, pl.ds(wid * span, span)])

    return pl.kernel(
        body,
        out_type=jax.ShapeDtypeStruct((3, B), jnp.float32),
        mesh=mesh,
        scratch_types=[
            pltpu.VMEM((m, gp_w * _LANES), jnp.int32),
            pltpu.VMEM((nper_w,), jnp.int32),
            pltpu.VMEM((per_w,), jnp.int32),
            pltpu.VMEM((per_w,), jnp.float32),
            pltpu.VMEM((3, gp_w * _LANES), jnp.float32),
            pltpu.SemaphoreType.DMA,
            pltpu.SemaphoreType.DMA,
        ],
    )


def _tc_reduce_call(stats3, B):
    inv = 1.0 / B
    ln2 = math.log(2.0)

    def body(st_ref, out_ref):
        r = st_ref[0]
        se = st_ref[1]
        d = st_ref[2]
        out_ref[7] = jnp.sum(jnp.log(se) + d) * inv
        w = ln2 / jnp.log(r + 2.0)
        for i, k in enumerate(_KS):
            hit = jnp.where(r < k, 1.0, 0.0)
            out_ref[2 * i] = jnp.sum(hit) * inv
            out_ref[2 * i + 1] = jnp.sum(hit * w) * inv
        out_ref[6] = jnp.sum(1.0 / (r + 1.0)) * inv

    return pl.pallas_call(
        body,
        out_shape=jax.ShapeDtypeStruct((8,), jnp.float32),
        out_specs=pl.BlockSpec(memory_space=pltpu.SMEM),
    )(stats3)


def kernel(scores, labels):
    B, C = scores.shape
    m = labels.shape[1]
    negs = jnp.asarray(_negs_const(B, C, m))
    labt = labels.T.reshape(-1)
    phys = (scores.reshape(B // 128, 128, C // 8, 8)
            .transpose(2, 0, 3, 1).reshape(-1))
    stats = _sc_stats_call(B, C, m)(phys, labt, negs)
    return _tc_reduce_call(stats.reshape(3, B // 128, 128), B)


# speedup vs baseline: 30.0177x; 1.0438x over previous
import functools
import math

import jax
import jax.numpy as jnp
import numpy as np
from jax import lax
from jax.experimental import pallas as pl
from jax.experimental.pallas import tpu as pltpu
from jax.experimental.pallas import tpu_sc as plsc

_NUM_NEG = 29
_NCAND = _NUM_NEG + 1
_KS = (1, 5, 10)
_LANES = 16
_NUM_CORES = 2
_NUM_SUBCORES = 16
_NW = _NUM_CORES * _NUM_SUBCORES
_CHUNK = 120


_LN2 = 0.6931471805599453
_SQRT2 = 1.4142135623730951

_ROT = ((13, 15, 26, 6), (17, 29, 16, 24))


def _ln(x):
    bits = lax.bitcast_convert_type(x, jnp.int32)
    e = ((bits >> 23) & 0xFF) - 127
    mant = lax.bitcast_convert_type((bits & 0x007FFFFF) | 0x3F800000,
                                    jnp.float32)
    big = mant > _SQRT2
    ione = jnp.ones((_LANES,), jnp.int32)
    izero = jnp.zeros((_LANES,), jnp.int32)
    mant = jnp.where(big, mant * 0.5, mant)
    e = (e + jnp.where(big, ione, izero)).astype(jnp.float32)
    s = (mant - 1.0) / (mant + 1.0)
    s2 = s * s
    p = 2.0 + s2 * (2.0 / 3.0 + s2 * (0.4 + s2 * (2.0 / 7.0)))
    return e * _LN2 + s * p


def _tf2x32(k1, k2, x1, x2):
    u32 = np.uint32
    ks = (u32(k1), u32(k2), u32(u32(k1) ^ u32(k2) ^ u32(0x1BD11BDA)))
    x0 = (x1 + ks[0]).astype(np.uint32)
    x1 = (x2 + ks[1]).astype(np.uint32)
    sched = ((0, 1, 2, 1), (1, 2, 0, 2), (0, 0, 1, 3), (1, 1, 2, 4),
             (0, 2, 0, 5))
    for g, a, b, inc in sched:
        for r in _ROT[g]:
            x0 = (x0 + x1).astype(np.uint32)
            x1 = x0 ^ ((x1 << u32(r)) | (x1 >> u32(32 - r)))
        x0 = (x0 + ks[a]).astype(np.uint32)
        x1 = (x1 + ks[b] + u32(inc)).astype(np.uint32)
    return x0, x1


@functools.lru_cache(maxsize=None)
def _negs_const(B, C, m):
    n = B * _NUM_NEG
    b1, b2 = _tf2x32(0, 42, np.zeros(2, np.uint32),
                     np.arange(2, dtype=np.uint32))
    cnt = np.arange(n, dtype=np.uint64)
    chi = (cnt >> np.uint64(32)).astype(np.uint32)
    clo = (cnt & np.uint64(0xFFFFFFFF)).astype(np.uint32)
    h1, h2 = _tf2x32(b1[0], b2[0], chi, clo)
    l1, l2 = _tf2x32(b1[1], b2[1], chi, clo)
    higher, lower = h1 ^ h2, l1 ^ l2
    span = np.uint32(C - m)
    mult = int(np.uint32(65536) % span)
    mult = np.uint32((mult * mult) % (1 << 32) % int(span))
    off = ((higher % span) * mult + (lower % span)).astype(np.uint32) % span
    negs = off.astype(np.int32).reshape(B, _NUM_NEG)
    return np.ascontiguousarray(
        negs.reshape(B // _LANES, _LANES, _NUM_NEG).transpose(0, 2, 1)
        .reshape(-1))


def _sc_stats_call(B, C, m):
    gp_w = B // _LANES // _NW
    per_w = gp_w * _NCAND * _LANES
    nper_w = gp_w * _NUM_NEG * _LANES
    n_chunks = per_w // _CHUNK
    assert per_w % _CHUNK == 0
    trow = (B // 128) * 1024
    mesh = plsc.VectorSubcoreMesh(
        core_axis_name="c", subcore_axis_name="s",
        num_cores=_NUM_CORES, num_subcores=_NUM_SUBCORES)

    def body(scores_hbm, labt_hbm, negs_hbm, out_hbm,
             lab_v, negs_v, idx_v, vals_v, stat_v, sem, gsem):
        wid = lax.axis_index("s") * _NUM_CORES + lax.axis_index("c")
        span = gp_w * _LANES
        lcopies = [
            pltpu.async_copy(labt_hbm.at[pl.ds(t * B + wid * span, span)],
                             lab_v.at[t], sem)
            for t in range(m)
        ]
        cn = pltpu.async_copy(
            negs_hbm.at[pl.ds(wid * nper_w, nper_w)], negs_v, sem)
        for cc in lcopies:
            cc.wait()
        cn.wait()
        for gl in range(gp_w):
            goff = gl * _LANES
            b = (wid * (gp_w * _LANES) + goff
                 + lax.iota(jnp.int32, _LANES))
            b_term = ((b >> 7) << 10) + (b & 127)
            svs = [lab_v[t, pl.ds(goff, _LANES)] for t in range(m)]
            x0c = svs[m - 1]
            for rnd in range(m):
                start = rnd & 1
                for t in range(start, m - 1, 2):
                    lo = jnp.minimum(svs[t], svs[t + 1])
                    hi = jnp.maximum(svs[t], svs[t + 1])
                    svs[t], svs[t + 1] = lo, hi
            one = jnp.ones((_LANES,), jnp.int32)
            zero = jnp.zeros((_LANES,), jnp.int32)
            nvs = [negs_v[pl.ds(gl * (_NUM_NEG * _LANES) + j * _LANES,
                                _LANES)] for j in range(_NUM_NEG)]
            for t in range(m):
                e = svs[t]
                for j in range(_NUM_NEG):
                    nvs[j] = nvs[j] + jnp.where(nvs[j] >= e, one, zero)
            cands = [x0c] + nvs
            ibase = gl * (_NCAND * _LANES)
            for j in range(_NCAND):
                c = cands[j]
                idx_v[pl.ds(ibase + j * _LANES, _LANES)] = (
                    ((c >> 3) * trow) + ((c & 7) << 7) + b_term)
        copies = [
            pltpu.async_copy(
                scores_hbm.at[idx_v.at[pl.ds(ch * _CHUNK, _CHUNK)]],
                vals_v.at[pl.ds(ch * _CHUNK, _CHUNK)], gsem)
            for ch in range(n_chunks)
        ]
        for cc in copies:
            cc.wait()
        parts = [jnp.zeros((_LANES,), jnp.float32) for _ in range(8)]
        for gl in range(gp_w):
            goff = gl * (_NCAND * _LANES)
            vs = [vals_v[pl.ds(goff + j * _LANES, _LANES)]
                  for j in range(_NCAND)]
            x0 = vs[0]
            mx = x0
            rank = jnp.zeros((_LANES,), jnp.float32)
            fone = jnp.ones((_LANES,), jnp.float32)
            fzero = jnp.zeros((_LANES,), jnp.float32)
            for j in range(1, _NCAND):
                rank = rank + jnp.where(vs[j] > x0, fone, fzero)
                mx = jnp.maximum(mx, vs[j])
            se = jnp.zeros((_LANES,), jnp.float32)
            for j in range(_NCAND):
                se = se + jnp.exp(vs[j] - mx)
            w = _LN2 / _ln(rank + 2.0)
            loss_t = _ln(se) + (mx - x0)
            mrr_t = fone / (rank + 1.0)
            for i, k in enumerate(_KS):
                hit = jnp.where(rank < k, fone, fzero)
                parts[2 * i] = parts[2 * i] + hit
                parts[2 * i + 1] = parts[2 * i + 1] + hit * w
            parts[6] = parts[6] + mrr_t
            parts[7] = parts[7] + loss_t
        for k in range(8):
            stat_v[pl.ds(k * _LANES, _LANES)] = parts[k]
        pltpu.sync_copy(stat_v, out_hbm.at[wid])

    return pl.kernel(
        body,
        out_type=jax.ShapeDtypeStruct((_NW, 8 * _LANES), jnp.float32),
        mesh=mesh,
        scratch_types=[
            pltpu.VMEM((m, gp_w * _LANES), jnp.int32),
            pltpu.VMEM((nper_w,), jnp.int32),
            pltpu.VMEM((per_w,), jnp.int32),
            pltpu.VMEM((per_w,), jnp.float32),
            pltpu.VMEM((8 * _LANES,), jnp.float32),
            pltpu.SemaphoreType.DMA,
            pltpu.SemaphoreType.DMA,
        ],
    )


def _tc_reduce_call(parts, B):
    inv = 1.0 / B

    def body(st_ref, out_ref):
        tot = jnp.sum(st_ref[...], axis=0, keepdims=True)
        for k in range(8):
            out_ref[k] = jnp.sum(tot[:, k * _LANES:(k + 1) * _LANES]) * inv

    return pl.pallas_call(
        body,
        out_shape=jax.ShapeDtypeStruct((8,), jnp.float32),
        out_specs=pl.BlockSpec(memory_space=pltpu.SMEM),
    )(parts)


def kernel(scores, labels):
    B, C = scores.shape
    m = labels.shape[1]
    negs = jnp.asarray(_negs_const(B, C, m))
    labt = labels.T.reshape(-1)
    phys = (scores.reshape(B // 128, 128, C // 8, 8)
            .transpose(2, 0, 3, 1).reshape(-1))
    parts = _sc_stats_call(B, C, m)(phys, labt, negs)
    return _tc_reduce_call(parts, B)


# speedup vs baseline: 30.4687x; 1.0150x over previous
import functools
import math

import jax
import jax.numpy as jnp
import numpy as np
from jax import lax
from jax.experimental import pallas as pl
from jax.experimental.pallas import tpu as pltpu
from jax.experimental.pallas import tpu_sc as plsc

_NUM_NEG = 29
_NCAND = _NUM_NEG + 1
_KS = (1, 5, 10)
_LANES = 16
_NUM_CORES = 2
_NUM_SUBCORES = 16
_NW = _NUM_CORES * _NUM_SUBCORES
_CHUNK = 120


_LN2 = 0.6931471805599453
_SQRT2 = 1.4142135623730951

_ROT = ((13, 15, 26, 6), (17, 29, 16, 24))


def _ln(x):
    bits = lax.bitcast_convert_type(x, jnp.int32)
    e = ((bits >> 23) & 0xFF) - 127
    mant = lax.bitcast_convert_type((bits & 0x007FFFFF) | 0x3F800000,
                                    jnp.float32)
    big = mant > _SQRT2
    ione = jnp.ones((_LANES,), jnp.int32)
    izero = jnp.zeros((_LANES,), jnp.int32)
    mant = jnp.where(big, mant * 0.5, mant)
    e = (e + jnp.where(big, ione, izero)).astype(jnp.float32)
    s = (mant - 1.0) / (mant + 1.0)
    s2 = s * s
    p = 2.0 + s2 * (2.0 / 3.0 + s2 * (0.4 + s2 * (2.0 / 7.0)))
    return e * _LN2 + s * p


def _tf2x32(k1, k2, x1, x2):
    u32 = np.uint32
    ks = (u32(k1), u32(k2), u32(u32(k1) ^ u32(k2) ^ u32(0x1BD11BDA)))
    x0 = (x1 + ks[0]).astype(np.uint32)
    x1 = (x2 + ks[1]).astype(np.uint32)
    sched = ((0, 1, 2, 1), (1, 2, 0, 2), (0, 0, 1, 3), (1, 1, 2, 4),
             (0, 2, 0, 5))
    for g, a, b, inc in sched:
        for r in _ROT[g]:
            x0 = (x0 + x1).astype(np.uint32)
            x1 = x0 ^ ((x1 << u32(r)) | (x1 >> u32(32 - r)))
        x0 = (x0 + ks[a]).astype(np.uint32)
        x1 = (x1 + ks[b] + u32(inc)).astype(np.uint32)
    return x0, x1


@functools.lru_cache(maxsize=None)
def _negs_const(B, C, m):
    n = B * _NUM_NEG
    b1, b2 = _tf2x32(0, 42, np.zeros(2, np.uint32),
                     np.arange(2, dtype=np.uint32))
    cnt = np.arange(n, dtype=np.uint64)
    chi = (cnt >> np.uint64(32)).astype(np.uint32)
    clo = (cnt & np.uint64(0xFFFFFFFF)).astype(np.uint32)
    h1, h2 = _tf2x32(b1[0], b2[0], chi, clo)
    l1, l2 = _tf2x32(b1[1], b2[1], chi, clo)
    higher, lower = h1 ^ h2, l1 ^ l2
    span = np.uint32(C - m)
    mult = int(np.uint32(65536) % span)
    mult = np.uint32((mult * mult) % (1 << 32) % int(span))
    off = ((higher % span) * mult + (lower % span)).astype(np.uint32) % span
    negs = off.astype(np.int32).reshape(B, _NUM_NEG)
    return np.ascontiguousarray(
        negs.reshape(B // _LANES, _LANES, _NUM_NEG).transpose(0, 2, 1)
        .reshape(-1))


def _sc_stats_call(B, C, m):
    gp_w = B // _LANES // _NW
    per_w = gp_w * _NCAND * _LANES
    nper_w = gp_w * _NUM_NEG * _LANES
    n_chunks = per_w // _CHUNK
    assert per_w % _CHUNK == 0
    trow = (B // 128) * 1024
    mesh = plsc.VectorSubcoreMesh(
        core_axis_name="c", subcore_axis_name="s",
        num_cores=_NUM_CORES, num_subcores=_NUM_SUBCORES)

    g_chunks = (_NCAND * _LANES) // _CHUNK
    assert (_NCAND * _LANES) % _CHUNK == 0

    def body(scores_hbm, labt_hbm, negs_hbm, out_hbm,
             lab_v, negs_v, idx_v, vals_v, stat_v, sem, gsem):
        wid = lax.axis_index("s") * _NUM_CORES + lax.axis_index("c")
        span = gp_w * _LANES
        lcopies = [
            pltpu.async_copy(labt_hbm.at[pl.ds(t * B + wid * span, span)],
                             lab_v.at[t], sem)
            for t in range(m)
        ]
        cn = pltpu.async_copy(
            negs_hbm.at[pl.ds(wid * nper_w, nper_w)], negs_v, sem)
        for cc in lcopies:
            cc.wait()
        cn.wait()
        copies = []
        for gl in range(gp_w):
            goff = gl * _LANES
            b = (wid * span + goff + lax.iota(jnp.int32, _LANES))
            b_term = ((b >> 7) << 10) + (b & 127)
            svs = [lab_v[t, pl.ds(goff, _LANES)] for t in range(m)]
            x0c = svs[m - 1]
            for rnd in range(m):
                for t in range(rnd & 1, m - 1, 2):
                    lo = jnp.minimum(svs[t], svs[t + 1])
                    hi = jnp.maximum(svs[t], svs[t + 1])
                    svs[t], svs[t + 1] = lo, hi
            one = jnp.ones((_LANES,), jnp.int32)
            zero = jnp.zeros((_LANES,), jnp.int32)
            nvs = [negs_v[pl.ds(gl * (_NUM_NEG * _LANES) + j * _LANES,
                                _LANES)] for j in range(_NUM_NEG)]
            for t in range(m):
                e = svs[t]
                for j in range(_NUM_NEG):
                    nvs[j] = nvs[j] + jnp.where(nvs[j] >= e, one, zero)
            cands = [x0c] + nvs
            ibase = gl * (_NCAND * _LANES)
            for j in range(_NCAND):
                c = cands[j]
                idx_v[pl.ds(ibase + j * _LANES, _LANES)] = (
                    ((c >> 3) * trow) + ((c & 7) << 7) + b_term)
            for ch in range(g_chunks):
                off = ibase + ch * _CHUNK
                copies.append(pltpu.async_copy(
                    scores_hbm.at[idx_v.at[pl.ds(off, _CHUNK)]],
                    vals_v.at[pl.ds(off, _CHUNK)], gsem))
        for cc in copies:
            cc.wait()
        parts = [jnp.zeros((_LANES,), jnp.float32) for _ in range(8)]
        for gl in range(gp_w):
            goff = gl * (_NCAND * _LANES)
            vs = [vals_v[pl.ds(goff + j * _LANES, _LANES)]
                  for j in range(_NCAND)]
            x0 = vs[0]
            mx = x0
            rank = jnp.zeros((_LANES,), jnp.float32)
            fone = jnp.ones((_LANES,), jnp.float32)
            fzero = jnp.zeros((_LANES,), jnp.float32)
            for j in range(1, _NCAND):
                rank = rank + jnp.where(vs[j] > x0, fone, fzero)
                mx = jnp.maximum(mx, vs[j])
            se = jnp.zeros((_LANES,), jnp.float32)
            for j in range(_NCAND):
                se = se + jnp.exp(vs[j] - mx)
            w = _LN2 / _ln(rank + 2.0)
            loss_t = _ln(se) + (mx - x0)
            mrr_t = fone / (rank + 1.0)
            for i, k in enumerate(_KS):
                hit = jnp.where(rank < k, fone, fzero)
                parts[2 * i] = parts[2 * i] + hit
                parts[2 * i + 1] = parts[2 * i + 1] + hit * w
            parts[6] = parts[6] + mrr_t
            parts[7] = parts[7] + loss_t
        for k in range(8):
            stat_v[pl.ds(k * _LANES, _LANES)] = parts[k]
        pltpu.sync_copy(stat_v, out_hbm.at[wid])

    return pl.kernel(
        body,
        out_type=jax.ShapeDtypeStruct((_NW, 8 * _LANES), jnp.float32),
        mesh=mesh,
        scratch_types=[
            pltpu.VMEM((m, gp_w * _LANES), jnp.int32),
            pltpu.VMEM((nper_w,), jnp.int32),
            pltpu.VMEM((per_w,), jnp.int32),
            pltpu.VMEM((per_w,), jnp.float32),
            pltpu.VMEM((8 * _LANES,), jnp.float32),
            pltpu.SemaphoreType.DMA,
            pltpu.SemaphoreType.DMA,
        ],
    )


def _tc_reduce_call(parts, B):
    inv = 1.0 / B

    def body(st_ref, out_ref):
        tot = jnp.sum(st_ref[...], axis=0, keepdims=True)
        for k in range(8):
            out_ref[k] = jnp.sum(tot[:, k * _LANES:(k + 1) * _LANES]) * inv

    return pl.pallas_call(
        body,
        out_shape=jax.ShapeDtypeStruct((8,), jnp.float32),
        out_specs=pl.BlockSpec(memory_space=pltpu.SMEM),
    )(parts)


def kernel(scores, labels):
    B, C = scores.shape
    m = labels.shape[1]
    negs = jnp.asarray(_negs_const(B, C, m))
    labt = labels.T.reshape(-1)
    phys = (scores.reshape(B // 128, 128, C // 8, 8)
            .transpose(2, 0, 3, 1).reshape(-1))
    parts = _sc_stats_call(B, C, m)(phys, labt, negs)
    return _tc_reduce_call(parts, B)


# speedup vs baseline: 30.8998x; 1.0141x over previous
import functools
import math

import jax
import jax.numpy as jnp
import numpy as np
from jax import lax
from jax.experimental import pallas as pl
from jax.experimental.pallas import tpu as pltpu
from jax.experimental.pallas import tpu_sc as plsc

_NUM_NEG = 29
_NCAND = _NUM_NEG + 1
_KS = (1, 5, 10)
_LANES = 16
_NUM_CORES = 2
_NUM_SUBCORES = 16
_NW = _NUM_CORES * _NUM_SUBCORES
_CHUNK = 120


_LN2 = 0.6931471805599453
_SQRT2 = 1.4142135623730951

_ROT = ((13, 15, 26, 6), (17, 29, 16, 24))


def _ln(x):
    bits = lax.bitcast_convert_type(x, jnp.int32)
    e = ((bits >> 23) & 0xFF) - 127
    mant = lax.bitcast_convert_type((bits & 0x007FFFFF) | 0x3F800000,
                                    jnp.float32)
    big = mant > _SQRT2
    ione = jnp.ones((_LANES,), jnp.int32)
    izero = jnp.zeros((_LANES,), jnp.int32)
    mant = jnp.where(big, mant * 0.5, mant)
    e = (e + jnp.where(big, ione, izero)).astype(jnp.float32)
    s = (mant - 1.0) / (mant + 1.0)
    s2 = s * s
    p = 2.0 + s2 * (2.0 / 3.0 + s2 * (0.4 + s2 * (2.0 / 7.0)))
    return e * _LN2 + s * p


def _tf2x32(k1, k2, x1, x2):
    u32 = np.uint32
    ks = (u32(k1), u32(k2), u32(u32(k1) ^ u32(k2) ^ u32(0x1BD11BDA)))
    x0 = (x1 + ks[0]).astype(np.uint32)
    x1 = (x2 + ks[1]).astype(np.uint32)
    sched = ((0, 1, 2, 1), (1, 2, 0, 2), (0, 0, 1, 3), (1, 1, 2, 4),
             (0, 2, 0, 5))
    for g, a, b, inc in sched:
        for r in _ROT[g]:
            x0 = (x0 + x1).astype(np.uint32)
            x1 = x0 ^ ((x1 << u32(r)) | (x1 >> u32(32 - r)))
        x0 = (x0 + ks[a]).astype(np.uint32)
        x1 = (x1 + ks[b] + u32(inc)).astype(np.uint32)
    return x0, x1


@functools.lru_cache(maxsize=None)
def _negs_const(B, C, m):
    n = B * _NUM_NEG
    b1, b2 = _tf2x32(0, 42, np.zeros(2, np.uint32),
                     np.arange(2, dtype=np.uint32))
    cnt = np.arange(n, dtype=np.uint64)
    chi = (cnt >> np.uint64(32)).astype(np.uint32)
    clo = (cnt & np.uint64(0xFFFFFFFF)).astype(np.uint32)
    h1, h2 = _tf2x32(b1[0], b2[0], chi, clo)
    l1, l2 = _tf2x32(b1[1], b2[1], chi, clo)
    higher, lower = h1 ^ h2, l1 ^ l2
    span = np.uint32(C - m)
    mult = int(np.uint32(65536) % span)
    mult = np.uint32((mult * mult) % (1 << 32) % int(span))
    off = ((higher % span) * mult + (lower % span)).astype(np.uint32) % span
    negs = off.astype(np.int32).reshape(B, _NUM_NEG)
    return np.ascontiguousarray(
        negs.reshape(B // _LANES, _LANES, _NUM_NEG).transpose(0, 2, 1)
        .reshape(-1))


def _sc_stats_call(B, C, m):
    gp_w = B // _LANES // _NW
    per_w = gp_w * _NCAND * _LANES
    nper_w = gp_w * _NUM_NEG * _LANES
    n_chunks = per_w // _CHUNK
    assert per_w % _CHUNK == 0
    trow = (B // 128) * 1024
    mesh = plsc.VectorSubcoreMesh(
        core_axis_name="c", subcore_axis_name="s",
        num_cores=_NUM_CORES, num_subcores=_NUM_SUBCORES)

    g_chunks = (_NCAND * _LANES) // _CHUNK
    assert (_NCAND * _LANES) % _CHUNK == 0

    def body(scores_hbm, labt_hbm, negs_hbm, out_hbm,
             lab_v, negs_v, idx_v, vals_v, stat_v, sem, gsem):
        wid = lax.axis_index("s") * _NUM_CORES + lax.axis_index("c")
        span = gp_w * _LANES
        blk = wid >> 2
        sub = (wid & 3) * span
        cl = pltpu.async_copy(labt_hbm.at[:, pl.ds(blk * 128, 128)],
                              lab_v, sem)
        cn = pltpu.async_copy(
            negs_hbm.at[pl.ds(wid * nper_w, nper_w)], negs_v, sem)
        cl.wait()
        cn.wait()
        copies = []
        for gl in range(gp_w):
            goff = gl * _LANES
            b = (wid * span + goff + lax.iota(jnp.int32, _LANES))
            b_term = ((b >> 7) << 10) + (b & 127)
            svs = [lab_v[t, pl.ds(sub + goff, _LANES)] for t in range(m)]
            x0c = svs[m - 1]
            for rnd in range(m):
                for t in range(rnd & 1, m - 1, 2):
                    lo = jnp.minimum(svs[t], svs[t + 1])
                    hi = jnp.maximum(svs[t], svs[t + 1])
                    svs[t], svs[t + 1] = lo, hi
            one = jnp.ones((_LANES,), jnp.int32)
            zero = jnp.zeros((_LANES,), jnp.int32)
            nvs = [negs_v[pl.ds(gl * (_NUM_NEG * _LANES) + j * _LANES,
                                _LANES)] for j in range(_NUM_NEG)]
            for t in range(m):
                e = svs[t]
                for j in range(_NUM_NEG):
                    nvs[j] = nvs[j] + jnp.where(nvs[j] >= e, one, zero)
            cands = [x0c] + nvs
            ibase = gl * (_NCAND * _LANES)
            for j in range(_NCAND):
                c = cands[j]
                idx_v[pl.ds(ibase + j * _LANES, _LANES)] = (
                    ((c >> 3) * trow) + ((c & 7) << 7) + b_term)
            for ch in range(g_chunks):
                off = ibase + ch * _CHUNK
                copies.append(pltpu.async_copy(
                    scores_hbm.at[idx_v.at[pl.ds(off, _CHUNK)]],
                    vals_v.at[pl.ds(off, _CHUNK)], gsem))
        for cc in copies:
            cc.wait()
        parts = [jnp.zeros((_LANES,), jnp.float32) for _ in range(8)]
        for gl in range(gp_w):
            goff = gl * (_NCAND * _LANES)
            vs = [vals_v[pl.ds(goff + j * _LANES, _LANES)]
                  for j in range(_NCAND)]
            x0 = vs[0]
            mx = x0
            rank = jnp.zeros((_LANES,), jnp.float32)
            fone = jnp.ones((_LANES,), jnp.float32)
            fzero = jnp.zeros((_LANES,), jnp.float32)
            for j in range(1, _NCAND):
                rank = rank + jnp.where(vs[j] > x0, fone, fzero)
                mx = jnp.maximum(mx, vs[j])
            se = jnp.zeros((_LANES,), jnp.float32)
            for j in range(_NCAND):
                se = se + jnp.exp(vs[j] - mx)
            w = _LN2 / _ln(rank + 2.0)
            loss_t = _ln(se) + (mx - x0)
            mrr_t = fone / (rank + 1.0)
            for i, k in enumerate(_KS):
                hit = jnp.where(rank < k, fone, fzero)
                parts[2 * i] = parts[2 * i] + hit
                parts[2 * i + 1] = parts[2 * i + 1] + hit * w
            parts[6] = parts[6] + mrr_t
            parts[7] = parts[7] + loss_t
        for k in range(8):
            stat_v[pl.ds(k * _LANES, _LANES)] = parts[k]
        pltpu.sync_copy(stat_v, out_hbm.at[wid])

    return pl.kernel(
        body,
        out_type=jax.ShapeDtypeStruct((_NW, 8 * _LANES), jnp.float32),
        mesh=mesh,
        scratch_types=[
            pltpu.VMEM((m, 128), jnp.int32),
            pltpu.VMEM((nper_w,), jnp.int32),
            pltpu.VMEM((per_w,), jnp.int32),
            pltpu.VMEM((per_w,), jnp.float32),
            pltpu.VMEM((8 * _LANES,), jnp.float32),
            pltpu.SemaphoreType.DMA,
            pltpu.SemaphoreType.DMA,
        ],
    )


def _tc_reduce_call(parts, B):
    inv = 1.0 / B

    def body(st_ref, out_ref):
        tot = jnp.sum(st_ref[...], axis=0, keepdims=True)
        for k in range(8):
            out_ref[k] = jnp.sum(tot[:, k * _LANES:(k + 1) * _LANES]) * inv

    return pl.pallas_call(
        body,
        out_shape=jax.ShapeDtypeStruct((8,), jnp.float32),
        out_specs=pl.BlockSpec(memory_space=pltpu.SMEM),
    )(parts)


def kernel(scores, labels):
    B, C = scores.shape
    m = labels.shape[1]
    negs = jnp.asarray(_negs_const(B, C, m))
    labt = labels.T
    phys = (scores.reshape(B // 128, 128, C // 8, 8)
            .transpose(2, 0, 3, 1).reshape(-1))
    parts = _sc_stats_call(B, C, m)(phys, labt, negs)
    return _tc_reduce_call(parts, B)
